# Initial kernel scaffold; baseline (speedup 1.0000x reference)
import jax
import jax.numpy as jnp
from jax.experimental import pallas as pl


def kernel(pointcloud, params, rand_idx):
    raise NotImplementedError("write your pallas kernel here")


# speedup vs baseline: 10.0324x; 10.0324x over previous
import functools
import jax
import jax.numpy as jnp
from jax import lax
from jax.experimental import pallas as pl
from jax.experimental.pallas import tpu as pltpu

K_NN = 32
DILATION = 2
NSAMPLE = 512
EPS = 1e-5

B = 2
N = 32768
S = NSAMPLE
BS = B * S
P = BS * K_NN
PB = 4096
NBLK = P // PB
SB = PB // K_NN

_f32 = jnp.float32
_bf16 = jnp.bfloat16


def _mm(x, w):
    return lax.dot_general(
        x.astype(_bf16), w.astype(_bf16),
        (((1,), (1,)), ((), ())), preferred_element_type=_f32)


def _norm_relu(x, st, g, b):
    m = st[0:1, :] / P
    v = st[1:2, :] / P - m * m
    return jnp.maximum((x - m) / jnp.sqrt(v + EPS) * g + b, 0.0)


def _colstats(y):
    return jnp.concatenate([jnp.sum(y, 0, keepdims=True),
                            jnp.sum(y * y, 0, keepdims=True)], axis=0)


def _stage_body(first, x_ref, st_ref, g_ref, b_ref, w_ref,
                y_ref, so_ref, acc_ref):
    x = x_ref[...]
    if first:
        xn = jnp.concatenate([x[:, 0:3], jnp.sqrt(x[:, 3:4] + 1e-12)], axis=1)
    else:
        xn = _norm_relu(x, st_ref[...], g_ref[...], b_ref[...])
    y = _mm(xn, w_ref[...])
    y_ref[...] = y
    i = pl.program_id(0)

    @pl.when(i == 0)
    def _():
        acc_ref[...] = jnp.zeros_like(acc_ref)

    acc_ref[...] += _colstats(y)

    @pl.when(i == NBLK - 1)
    def _():
        so_ref[...] = acc_ref[...]


def _stage(x, st, g, b, w, first=False):
    cin = x.shape[1]
    co = w.shape[0]
    body = functools.partial(_stage_body, first)
    return pl.pallas_call(
        body,
        grid=(NBLK,),
        in_specs=[
            pl.BlockSpec((PB, cin), lambda i: (i, 0)),
            pl.BlockSpec((2, cin), lambda i: (0, 0)),
            pl.BlockSpec((1, cin), lambda i: (0, 0)),
            pl.BlockSpec((1, cin), lambda i: (0, 0)),
            pl.BlockSpec((co, cin), lambda i: (0, 0)),
        ],
        out_specs=[
            pl.BlockSpec((PB, co), lambda i: (i, 0)),
            pl.BlockSpec((2, co), lambda i: (0, 0)),
        ],
        out_shape=[
            jax.ShapeDtypeStruct((P, co), _f32),
            jax.ShapeDtypeStruct((2, co), _f32),
        ],
        scratch_shapes=[pltpu.VMEM((2, co), _f32)],
    )(x, st, g, b, w)


def _attn_body(e_ref, st_ref, g_ref, b_ref, cl_ref, xs_ref, w4c_ref,
               aw_ref, kp_ref, gcf_ref, t2_ref):
    en = _norm_relu(e_ref[...], st_ref[...], g_ref[...], b_ref[...])
    er = en.reshape(SB, K_NN, 256)
    x1 = jnp.max(er, axis=2)
    xm = jnp.max(x1, axis=1, keepdims=True)
    ex = jnp.exp(x1 - xm)
    aw = ex / jnp.sum(ex, axis=1, keepdims=True)
    aw_ref[...] = aw
    gcf_ref[...] = jnp.sum(er * aw[:, :, None], axis=1)
    rel = cl_ref[...][:, 0:3].reshape(SB, K_NN, 3)
    kp_ref[...] = jnp.sum(rel * aw[:, :, None], axis=1) + xs_ref[...]
    t2 = _mm(en, w4c_ref[...])
    t2_ref[...] = (t2.reshape(SB, K_NN, 128)
                   * aw[:, :, None]).reshape(PB, 128)


def _attn(e3, st3, g, b, cluster, xs_flat, w4c):
    return pl.pallas_call(
        _attn_body,
        grid=(NBLK,),
        in_specs=[
            pl.BlockSpec((PB, 256), lambda i: (i, 0)),
            pl.BlockSpec((2, 256), lambda i: (0, 0)),
            pl.BlockSpec((1, 256), lambda i: (0, 0)),
            pl.BlockSpec((1, 256), lambda i: (0, 0)),
            pl.BlockSpec((PB, 4), lambda i: (i, 0)),
            pl.BlockSpec((SB, 3), lambda i: (i, 0)),
            pl.BlockSpec((128, 256), lambda i: (0, 0)),
        ],
        out_specs=[
            pl.BlockSpec((SB, K_NN), lambda i: (i, 0)),
            pl.BlockSpec((SB, 3), lambda i: (i, 0)),
            pl.BlockSpec((SB, 256), lambda i: (i, 0)),
            pl.BlockSpec((PB, 128), lambda i: (i, 0)),
        ],
        out_shape=[
            jax.ShapeDtypeStruct((BS, K_NN), _f32),
            jax.ShapeDtypeStruct((BS, 3), _f32),
            jax.ShapeDtypeStruct((BS, 256), _f32),
            jax.ShapeDtypeStruct((P, 128), _f32),
        ],
    )(e3, st3, g, b, cluster, xs_flat, w4c)


def _bn_relu_full(y, g, b):
    m = jnp.mean(y, 0, keepdims=True)
    yc = y - m
    v = jnp.mean(yc * yc, 0, keepdims=True)
    return jnp.maximum(yc / jnp.sqrt(v + EPS) * g + b, 0.0)


def _mlp_body(gcf_ref, w1, b1, g1, bb1, w2, b2, g2, bb2, w3, b3, sig_ref):
    h = _bn_relu_full(_mm(gcf_ref[...], w1[...]) + b1[...],
                      g1[...], bb1[...])
    h = _bn_relu_full(_mm(h, w2[...]) + b2[...],
                      g2[...], bb2[...])
    w3r = w3[...].astype(_bf16).astype(_f32)
    h3 = jnp.sum(h.astype(_bf16).astype(_f32) * w3r,
                 axis=1, keepdims=True) + b3[...]
    sig_ref[...] = (jnp.maximum(h3, 0.0)
                    + jnp.log1p(jnp.exp(-jnp.abs(h3))) + 0.001)


def _mlp(gcf, p):
    return pl.pallas_call(
        _mlp_body,
        out_shape=jax.ShapeDtypeStruct((BS, 1), _f32),
    )(gcf, p['w_m1'], p['bias_m1'], p['g_m1'], p['b_m1'],
      p['w_m2'], p['bias_m2'], p['g_m2'], p['b_m2'],
      p['w_m3'], p['bias_m3'])


def _prep4_body(y3_ref, st_ref, g_ref, b_ref, w4b_ref, ymax_ref, t1_ref):
    yn = _norm_relu(y3_ref[...], st_ref[...], g_ref[...], b_ref[...])
    ymax_ref[...] = jnp.max(yn.reshape(SB, K_NN, 128), axis=1)
    t1_ref[...] = _mm(yn, w4b_ref[...])


def _prep4(y3, st, g, b, w4b):
    return pl.pallas_call(
        _prep4_body,
        grid=(NBLK,),
        in_specs=[
            pl.BlockSpec((PB, 128), lambda i: (i, 0)),
            pl.BlockSpec((2, 128), lambda i: (0, 0)),
            pl.BlockSpec((1, 128), lambda i: (0, 0)),
            pl.BlockSpec((1, 128), lambda i: (0, 0)),
            pl.BlockSpec((128, 128), lambda i: (0, 0)),
        ],
        out_specs=[
            pl.BlockSpec((SB, 128), lambda i: (i, 0)),
            pl.BlockSpec((PB, 128), lambda i: (i, 0)),
        ],
        out_shape=[
            jax.ShapeDtypeStruct((BS, 128), _f32),
            jax.ShapeDtypeStruct((P, 128), _f32),
        ],
    )(y3, st, g, b, w4b)


def _c4_body(t1_ref, t2_ref, ymax_ref, w4a_ref, y_ref, so_ref, acc_ref):
    z2 = _mm(ymax_ref[...], w4a_ref[...])
    y = (t1_ref[...] + t2_ref[...]
         + jnp.broadcast_to(z2[:, None, :],
                            (SB, K_NN, 128)).reshape(PB, 128))
    y_ref[...] = y
    i = pl.program_id(0)

    @pl.when(i == 0)
    def _():
        acc_ref[...] = jnp.zeros_like(acc_ref)

    acc_ref[...] += _colstats(y)

    @pl.when(i == NBLK - 1)
    def _():
        so_ref[...] = acc_ref[...]


def _c4(t1, t2, ymax, w4a):
    return pl.pallas_call(
        _c4_body,
        grid=(NBLK,),
        in_specs=[
            pl.BlockSpec((PB, 128), lambda i: (i, 0)),
            pl.BlockSpec((PB, 128), lambda i: (i, 0)),
            pl.BlockSpec((SB, 128), lambda i: (i, 0)),
            pl.BlockSpec((128, 128), lambda i: (0, 0)),
        ],
        out_specs=[
            pl.BlockSpec((PB, 128), lambda i: (i, 0)),
            pl.BlockSpec((2, 128), lambda i: (0, 0)),
        ],
        out_shape=[
            jax.ShapeDtypeStruct((P, 128), _f32),
            jax.ShapeDtypeStruct((2, 128), _f32),
        ],
        scratch_shapes=[pltpu.VMEM((2, 128), _f32)],
    )(t1, t2, ymax, w4a)


def _desc_body(c5_ref, st_ref, g_ref, b_ref, desc_ref):
    cn = _norm_relu(c5_ref[...], st_ref[...], g_ref[...], b_ref[...])
    desc_ref[...] = jnp.max(cn.reshape(SB, K_NN, 128), axis=1)


def _desc(c5, st, g, b):
    return pl.pallas_call(
        _desc_body,
        grid=(NBLK,),
        in_specs=[
            pl.BlockSpec((PB, 128), lambda i: (i, 0)),
            pl.BlockSpec((2, 128), lambda i: (0, 0)),
            pl.BlockSpec((1, 128), lambda i: (0, 0)),
            pl.BlockSpec((1, 128), lambda i: (0, 0)),
        ],
        out_specs=pl.BlockSpec((SB, 128), lambda i: (i, 0)),
        out_shape=jax.ShapeDtypeStruct((BS, 128), _f32),
    )(c5, st, g, b)


def _dense(cluster4, xs_flat, pin):
    p = {k: (v.reshape(1, -1) if v.ndim == 1 else v) for k, v in pin.items()}
    z4 = jnp.zeros((1, 4), _f32)
    e1, st1 = _stage(cluster4, jnp.zeros((2, 4), _f32),
                     z4, z4, p['w_c1'], first=True)
    e2, st2 = _stage(e1, st1, p['g_c1'], p['b_c1'], p['w_c2'])
    e3, st3 = _stage(e2, st2, p['g_c2'], p['b_c2'], p['w_c3'])
    w4 = p['w_c4']
    aw, kp, gcf, t2 = _attn(e3, st3, p['g_c3'], p['b_c3'],
                            cluster4, xs_flat, w4[:, 256:512])
    sig = _mlp(gcf, p)
    y1, sd1 = _stage(cluster4, jnp.zeros((2, 4), _f32),
                     z4, z4, p['w_d1'], first=True)
    y2, sd2 = _stage(y1, sd1, p['g_d1'], p['b_d1'], p['w_d2'])
    y3, sd3 = _stage(y2, sd2, p['g_d2'], p['b_d2'], p['w_d3'])
    ymax, t1 = _prep4(y3, sd3, p['g_d3'], p['b_d3'], w4[:, 128:256])
    c4, sc4 = _c4(t1, t2, ymax, w4[:, 0:128])
    c5, sc5 = _stage(c4, sc4, p['g_c4'], p['b_c4'], p['w_c5'])
    desc = _desc(c5, sc5, p['g_c5'], p['b_c5'])
    return kp, sig, desc


from jax.experimental.pallas import tpu_sc as plsc

DN = 8192
ROWS_PER_W = BS // 32
CAP = 512
INF = jnp.inf


def _k1a_body(xsT_ref, pcT_ref, d2_ref):
    a = xsT_ref[0]
    bb = pcT_ref[0]
    dot = lax.dot_general(a.astype(_bf16), bb.astype(_bf16),
                          (((0,), (0,)), ((), ())),
                          preferred_element_type=_f32)
    xs2 = jnp.sum(a * a, 0)[:, None]
    pc2 = jnp.sum(bb * bb, 0)[None, :]
    d2_ref[0] = (xs2 + pc2) - 2.0 * dot


def _k1a(xsT, pcT):
    return pl.pallas_call(
        _k1a_body,
        grid=(B, N // DN),
        in_specs=[
            pl.BlockSpec((1, 3, S), lambda b, n: (b, 0, 0)),
            pl.BlockSpec((1, 3, DN), lambda b, n: (b, 0, n)),
        ],
        out_specs=pl.BlockSpec((1, S, DN), lambda b, n: (b, 0, n)),
        out_shape=jax.ShapeDtypeStruct((B, S, N), _f32),
    )(xsT, pcT)


_GDN = lax.GatherDimensionNumbers(offset_dims=(), collapsed_slice_dims=(0,),
                                  start_index_map=(0,))


def _vgather(v, idx):
    return lax.gather(v, idx[:, None], _GDN, (1,),
                      mode=lax.GatherScatterMode.PROMISE_IN_BOUNDS)


def _sort16(k):
    ks, _ = plsc.sort_key_val(k, jnp.zeros((16,), jnp.int32))
    return ks


def _m16k(a, b):
    rb = lax.rev(b, (0,))
    return _sort16(jnp.minimum(a, rb)), _sort16(jnp.maximum(a, rb))


def _m16kv(ak, av, bk, bv):
    rbk = lax.rev(bk, (0,))
    rbv = lax.rev(bv, (0,))
    m = ak <= rbk
    lok = jnp.where(m, ak, rbk)
    lov = jnp.where(m, av, rbv)
    hik = jnp.where(m, rbk, ak)
    hiv = jnp.where(m, rbv, av)
    lok, lov = plsc.sort_key_val(lok, lov)
    hik, hiv = plsc.sort_key_val(hik, hiv)
    return lok, lov, hik, hiv


def _k1b_body(d2, pcx, pcy, pcz, xs8, out,
              dbuf0, dbuf1, mbuf, abk, abv, idxb, gx, gy, gz, xsb, outb,
              sem0, sem1, semg):
    wid = lax.axis_index("s") * 2 + lax.axis_index("c")
    row0 = wid * ROWS_PER_W
    base = (wid // 16) * N
    inf16 = jnp.full((16,), INF, _f32)
    io16 = lax.iota(jnp.int32, 16)
    full15 = jnp.full((16,), 15, jnp.int32)
    sems = (sem0, sem1)
    dbufs = (dbuf0, dbuf1)

    pltpu.async_copy(d2.at[row0], dbuf0, sem0)

    def pair_body(pr, carry):
        for b2 in range(2):
            r = row0 + pr * 2 + b2
            pltpu.make_async_copy(d2.at[r], dbufs[b2], sems[b2]).wait()
            nr = jnp.minimum(r + 1, BS - 1)
            pltpu.async_copy(d2.at[nr], dbufs[1 - b2], sems[1 - b2])
            db = dbufs[b2]

            for blk in range(8):
                def amin(j, m, _blk=blk):
                    for u in range(8):
                        m = jnp.minimum(
                            m, db[pl.ds((_blk * 256 + j * 8 + u) * 16, 16)])
                    return m
                mbuf[pl.ds(blk * 16, 16)] = lax.fori_loop(0, 32, amin, inf16)

            t0 = t1 = t2 = t3 = inf16
            for q in range(8):
                run = _sort16(mbuf[pl.ds(q * 16, 16)])
                t0, run = _m16k(t0, run)
                t1, run = _m16k(t1, run)
                t2, run = _m16k(t2, run)
                t3, run = _m16k(t3, run)
            tau = _vgather(t3, full15)

            def bscan(j, c):
                cnt, lanev = c
                for u in range(4):
                    v = db[pl.ds(j * 64 + u * 16, 16)]
                    msk = v <= tau
                    ones = jnp.where(msk, 1, 0).astype(jnp.int32)
                    cum = plsc.cumsum(ones)
                    pos = jnp.minimum(cnt + (cum - 1), CAP - 17)
                    plsc.store_scatter(abk, [pos], v, mask=msk)
                    plsc.store_scatter(abv, [pos], lanev, mask=msk)
                    cnt = cnt + _vgather(cum, full15)
                    lanev = lanev + 16
                return (cnt, lanev)
            cnt, _ = lax.fori_loop(0, N // 64, bscan,
                                   (jnp.zeros((16,), jnp.int32), io16))

            cnt_s = jnp.minimum(jnp.max(cnt), CAP - 16)
            abk[pl.ds(cnt_s, 16)] = inf16

            def sel(q, T):
                t0k, t0v, t1k, t1v, t2k, t2v, t3k, t3v = T
                rk, rv = plsc.sort_key_val(abk[pl.ds(q * 16, 16)],
                                           abv[pl.ds(q * 16, 16)])
                t0k, t0v, rk, rv = _m16kv(t0k, t0v, rk, rv)
                t1k, t1v, rk, rv = _m16kv(t1k, t1v, rk, rv)
                t2k, t2v, rk, rv = _m16kv(t2k, t2v, rk, rv)
                t3k, t3v, rk, rv = _m16kv(t3k, t3v, rk, rv)
                return (t0k, t0v, t1k, t1v, t2k, t2v, t3k, t3v)
            z16 = jnp.zeros((16,), jnp.int32)
            nv = lax.div(cnt_s + 15, 16)
            T = lax.fori_loop(0, nv, sel,
                              (inf16, z16, inf16, z16, inf16, z16,
                               inf16, z16))

            p16 = (2 * io16) & 15
            lo8 = io16 < 8
            ev0 = jnp.where(lo8, _vgather(T[1], p16), _vgather(T[3], p16))
            ev1 = jnp.where(lo8, _vgather(T[5], p16), _vgather(T[7], p16))
            idxb[pl.ds(0, 16)] = ev0 + base
            idxb[pl.ds(16, 16)] = ev1 + base

            pltpu.async_copy(pcx.at[idxb], gx, semg).wait()
            pltpu.async_copy(pcy.at[idxb], gy, semg).wait()
            pltpu.async_copy(pcz.at[idxb], gz, semg).wait()
            pltpu.sync_copy(xs8.at[pl.ds(r * 8, 8)], xsb.at[pl.ds(0, 8)])
            xsv = xsb[...]
            sx = _vgather(xsv, jnp.zeros((16,), jnp.int32))
            sy = _vgather(xsv, jnp.full((16,), 1, jnp.int32))
            sz = _vgather(xsv, jnp.full((16,), 2, jnp.int32))
            for h in range(2):
                rx = gx[pl.ds(h * 16, 16)] - sx
                ry = gy[pl.ds(h * 16, 16)] - sy
                rz = gz[pl.ds(h * 16, 16)] - sz
                ss = rx * rx + ry * ry + rz * rz
                ob = io16 * 4 + h * 64
                plsc.store_scatter(outb, [ob], rx)
                plsc.store_scatter(outb, [ob + 1], ry)
                plsc.store_scatter(outb, [ob + 2], rz)
                plsc.store_scatter(outb, [ob + 3], ss)
            pltpu.sync_copy(outb, out.at[pl.ds(r * 128, 128)])
        return carry

    lax.fori_loop(0, ROWS_PER_W // 2, pair_body, 0)
    pltpu.make_async_copy(d2.at[row0], dbuf0, sem0).wait()


def _k1b(d2_flat, pcx, pcy, pcz, xs8):
    mesh = plsc.VectorSubcoreMesh(core_axis_name="c", subcore_axis_name="s")
    kfn = functools.partial(
        pl.kernel,
        mesh=mesh,
        compiler_params=pltpu.CompilerParams(needs_layout_passes=False),
        out_type=jax.ShapeDtypeStruct((P * 4,), _f32),
        scratch_types=[
            pltpu.VMEM((N,), _f32),
            pltpu.VMEM((N,), _f32),
            pltpu.VMEM((128,), _f32),
            pltpu.VMEM((CAP,), _f32),
            pltpu.VMEM((CAP,), jnp.int32),
            pltpu.VMEM((32,), jnp.int32),
            pltpu.VMEM((32,), _f32),
            pltpu.VMEM((32,), _f32),
            pltpu.VMEM((32,), _f32),
            pltpu.VMEM((16,), _f32),
            pltpu.VMEM((128,), _f32),
            pltpu.SemaphoreType.DMA,
            pltpu.SemaphoreType.DMA,
            pltpu.SemaphoreType.DMA,
        ],
    )(_k1b_body)
    return kfn(d2_flat, pcx, pcy, pcz, xs8)


def kernel(pointcloud, params, rand_idx):
    pc = pointcloud
    xs = pc[:, rand_idx, :]
    xsT = xs.transpose(0, 2, 1)
    pcT = pc.transpose(0, 2, 1)
    d2 = _k1a(xsT, pcT).reshape(BS, N)
    pcx = pcT[:, 0, :].reshape(B * N)
    pcy = pcT[:, 1, :].reshape(B * N)
    pcz = pcT[:, 2, :].reshape(B * N)
    xs_flat = xs.reshape(BS, 3)
    xs8 = jnp.pad(xs_flat, ((0, 0), (0, 5))).reshape(-1)
    cluster4 = _k1b(d2, pcx, pcy, pcz, xs8).reshape(P, 4)
    kp, sig, desc = _dense(cluster4, xs_flat, params)
    keypoints = kp.reshape(B, S, 3).transpose(0, 2, 1)
    sigmas = sig.reshape(B, S)
    desc = desc.reshape(B, S, 128).transpose(0, 2, 1)
    return keypoints, sigmas, desc
